# baseline (speedup 1.0000x reference)
import functools

import jax
import jax.numpy as jnp
from jax import lax
from jax.experimental import pallas as pl
from jax.experimental.pallas import tpu as pltpu
from jax.experimental.pallas import tpu_sc as plsc

_D = 1024
_BATCH = 4
_SEQ = 8192
_NW = 32
_ROWS_PER_W = _SEQ // _NW
_R = 4
_NCHUNK = _ROWS_PER_W // _R
_NPHASE = 4
_LANES = 16
_DSLICES = _D // _LANES
_G = 8


def _pe_add_kernel(x_hbm, pe_hbm, out_hbm, pe_v, x_v, pe_sem, in_sem,
                   out_sem):
    cid = lax.axis_index("c")
    sid = lax.axis_index("s")
    wid = cid * 16 + sid
    row0 = wid * _ROWS_PER_W

    def start_pe(c, pb):
        pltpu.async_copy(pe_hbm.at[pl.ds(row0 + c * _R, _R)], pe_v.at[pb],
                         pe_sem)

    def wait_pe():
        pltpu.make_async_copy(pe_hbm.at[pl.ds(0, _R)], pe_v.at[0],
                              pe_sem).wait()

    def start_in(c, b, ph):
        pltpu.async_copy(x_hbm.at[b, pl.ds(row0 + c * _R, _R)],
                         x_v.at[ph, b], in_sem)

    def wait_in():
        pltpu.make_async_copy(x_hbm.at[0, pl.ds(0, _R)], x_v.at[0, 0],
                              in_sem).wait()

    def start_out(c, b, ph):
        pltpu.async_copy(x_v.at[ph, b],
                         out_hbm.at[b, pl.ds(row0 + c * _R, _R)], out_sem)

    def wait_out():
        pltpu.make_async_copy(x_v.at[0, 0],
                              out_hbm.at[0, pl.ds(0, _R)], out_sem).wait()

    def add_pass(ph):
        for r in range(_R):
            for g0 in range(0, _DSLICES, _G):
                vals = [pe_v[ph % 2, r, pl.ds((g0 + k) * _LANES, _LANES)]
                        for k in range(_G)]
                for k in range(_G):
                    for b in range(_BATCH):
                        plsc.addupdate(
                            x_v.at[ph, b, r,
                                   pl.ds((g0 + k) * _LANES, _LANES)],
                            vals[k])

    start_pe(0, 0)
    for c in (0, 1):
        for b in range(_BATCH):
            start_in(c, b, c)

    def chunk_quad(c4, _):
        for u in range(_NPHASE):
            c = 4 * c4 + u
            wait_pe()
            if u < _NPHASE - 1:
                start_pe(c + 1, (u + 1) % 2)
            else:
                @pl.when(c4 != _NCHUNK // _NPHASE - 1)
                def _():
                    start_pe(c + 1, (u + 1) % 2)
            for b in range(_BATCH):
                wait_in()
            add_pass(u)
            for b in range(_BATCH):
                start_out(c, b, u)
            if u < 2:
                @pl.when(c4 != 0)
                def _():
                    for b in range(_BATCH):
                        wait_out()
                for b in range(_BATCH):
                    start_in(c + 2, b, (u + 2) % _NPHASE)
            else:
                @pl.when(c4 != _NCHUNK // _NPHASE - 1)
                def _():
                    for b in range(_BATCH):
                        wait_out()
                    for b in range(_BATCH):
                        start_in(c + 2, b, (u + 2) % _NPHASE)
        return 0

    lax.fori_loop(0, _NCHUNK // _NPHASE, chunk_quad, 0)
    for _ in range(4 * _BATCH):
        wait_out()


@jax.jit
def kernel(x, pe_weight):
    mesh = plsc.VectorSubcoreMesh(core_axis_name="c", subcore_axis_name="s")
    run = functools.partial(
        pl.kernel,
        mesh=mesh,
        out_type=jax.ShapeDtypeStruct((_BATCH, _SEQ, _D), jnp.float32),
        scratch_types=[
            pltpu.VMEM((2, _R, _D), jnp.float32),
            pltpu.VMEM((_NPHASE, _BATCH, _R, _D), jnp.float32),
            pltpu.SemaphoreType.DMA,
            pltpu.SemaphoreType.DMA,
            pltpu.SemaphoreType.DMA,
        ],
    )(_pe_add_kernel)
    return run(x, pe_weight)


# speedup vs baseline: 1.5111x; 1.5111x over previous
import functools

import jax
import jax.numpy as jnp
from jax import lax
from jax.experimental import pallas as pl
from jax.experimental.pallas import tpu as pltpu
from jax.experimental.pallas import tpu_sc as plsc

_D = 1024
_BATCH = 4
_SEQ = 8192
_NW = 32
_ROWS_PER_W = _SEQ // _NW
_R = 8
_NCHUNK = _ROWS_PER_W // _R
_LANES = 16
_DSLICES = _D // _LANES
_G = 4


def _pe_add_kernel(x_hbm, pe_hbm, out_hbm, pe_v, x_v, pe_sem, in_sem, out_sem):
    cid = lax.axis_index("c")
    sid = lax.axis_index("s")
    wid = cid * 16 + sid
    row0 = wid * _ROWS_PER_W

    def start_pe(c, buf):
        pltpu.async_copy(pe_hbm.at[pl.ds(row0 + c * _R, _R)], pe_v.at[buf],
                         pe_sem)

    def start_in(c, b, ph):
        pltpu.async_copy(x_hbm.at[b, pl.ds(row0 + c * _R, _R)],
                         x_v.at[ph, b], in_sem)

    def wait_pe():
        pltpu.make_async_copy(pe_hbm.at[pl.ds(0, _R)], pe_v.at[0],
                              pe_sem).wait()

    def wait_in():
        pltpu.make_async_copy(pe_hbm.at[pl.ds(0, _R)], x_v.at[0, 0],
                              in_sem).wait()

    def wait_out():
        pltpu.make_async_copy(x_v.at[0, 0], out_hbm.at[0, pl.ds(0, _R)],
                              out_sem).wait()

    def add_fused(ph, pb):
        def body(r, _):
            for g0 in range(0, _DSLICES, _G):
                vals = [pe_v[pb, r, pl.ds((g0 + k) * _LANES, _LANES)]
                        for k in range(_G)]
                for k in range(_G):
                    for b in range(_BATCH):
                        plsc.addupdate(
                            x_v.at[ph, b, r,
                                   pl.ds((g0 + k) * _LANES, _LANES)],
                            vals[k])
            return 0

        lax.fori_loop(0, _R, body, 0)

    start_pe(0, 0)
    for c in (0, 1):
        for b in range(_BATCH):
            start_in(c, b, c)

    def chunk_pair(c2, _):
        for cc in (0, 1):
            c = 2 * c2 + cc
            wait_pe()
            if cc == 0:
                start_pe(c + 1, 1)
            else:
                @pl.when(c2 != _NCHUNK // 2 - 1)
                def _():
                    start_pe(c + 1, 0)
            for b in range(_BATCH):
                wait_in()
            if cc == 0:
                @pl.when(c2 != 0)
                def _():
                    for b in range(_BATCH):
                        wait_out()
                    for b in range(_BATCH):
                        start_in(c + 1, b, 1)
            else:
                @pl.when(c2 != _NCHUNK // 2 - 1)
                def _():
                    for b in range(_BATCH):
                        wait_out()
                    for b in range(_BATCH):
                        start_in(c + 1, b, 0)
            add_fused(cc, cc)
            for b in range(_BATCH):
                pltpu.async_copy(x_v.at[cc, b],
                                 out_hbm.at[b, pl.ds(row0 + c * _R, _R)],
                                 out_sem)
        return 0

    lax.fori_loop(0, _NCHUNK // 2, chunk_pair, 0)
    for _ in range(2 * _BATCH):
        wait_out()


@jax.jit
def kernel(x, pe_weight):
    mesh = plsc.VectorSubcoreMesh(core_axis_name="c", subcore_axis_name="s")
    run = functools.partial(
        pl.kernel,
        mesh=mesh,
        out_type=jax.ShapeDtypeStruct((_BATCH, _SEQ, _D), jnp.float32),
        scratch_types=[
            pltpu.VMEM((2, _R, _D), jnp.float32),
            pltpu.VMEM((2, _BATCH, _R, _D), jnp.float32),
            pltpu.SemaphoreType.DMA,
            pltpu.SemaphoreType.DMA,
            pltpu.SemaphoreType.DMA,
        ],
    )(_pe_add_kernel)
    return run(x, pe_weight)
